# baseline (speedup 1.0000x reference)
import functools

import jax
import jax.numpy as jnp
from jax import lax
from jax.experimental import pallas as pl
from jax.experimental.pallas import tpu as pltpu
from jax.experimental.pallas import tpu_sc as plsc

N = 100000
E = 3200000
D = 16

NC = 2
NS = 16
NW = NC * NS

LANE = 128
N_PAD = 100352
ROWS_PER_TILE = N_PAD // NS
E_PER_TILE = E // NW
HCHUNK = 1024
H_PAIRS = 48
H_TAIL = (1024, 512, 128, 32)
BLK = 2048
GRID = N_PAD // BLK

_mesh = plsc.VectorSubcoreMesh(
    core_axis_name="c", subcore_axis_name="s", num_cores=NC, num_subcores=NS)


@functools.partial(
    pl.kernel,
    out_type=jax.ShapeDtypeStruct((NC, N_PAD), jnp.float32),
    mesh=_mesh,
    scratch_types=[
        pltpu.VMEM_SHARED((N_PAD,), jnp.float32),
        pltpu.VMEM((2, HCHUNK), jnp.int32),
        pltpu.VMEM((HCHUNK,), jnp.float32),
        pltpu.VMEM((ROWS_PER_TILE // 8,), jnp.float32),
        pltpu.SemaphoreType.DMA,
    ],
    compiler_params=pltpu.CompilerParams(use_tc_tiling_on_sc=False),
)
def _sc_degree(edges_hbm, degp_hbm, deg_sp, idx_v, ones_v, zbuf, ssem):
    c = lax.axis_index("c")
    s = lax.axis_index("s")
    wid = s * NC + c
    zc = ROWS_PER_TILE // 8

    def zfill(i, carry):
        zbuf[pl.ds(i * 16, 16)] = jnp.zeros((16,), jnp.float32)
        return carry

    lax.fori_loop(0, zc // 16, zfill, 0)

    def ofill(i, carry):
        ones_v[pl.ds(i * 16, 16)] = jnp.ones((16,), jnp.float32)
        return carry

    lax.fori_loop(0, HCHUNK // 16, ofill, 0)
    for q in range(8):
        pltpu.sync_copy(zbuf,
                        deg_sp.at[pl.ds(s * ROWS_PER_TILE + q * zc, zc)])
    plsc.subcore_barrier()

    base = wid * E_PER_TILE

    def stage(b, ch):
        pltpu.sync_copy(edges_hbm.at[1, pl.ds(base + ch * HCHUNK, HCHUNK)],
                        idx_v.at[b])

    def fire(b):
        pltpu.async_copy(ones_v, deg_sp.at[idx_v.at[b]], ssem, add=True)

    def drain(b):
        pltpu.make_async_copy(ones_v, deg_sp.at[idx_v.at[b]], ssem).wait()

    stage(0, 0)

    def body(p, carry):
        fire(0)
        stage(1, 2 * p + 1)
        drain(0)
        fire(1)

        @pl.when(p + 1 < H_PAIRS)
        def _():
            stage(0, 2 * p + 2)
        drain(1)
        return carry

    lax.fori_loop(0, H_PAIRS, body, 0)
    off = 2 * H_PAIRS * HCHUNK
    for tl in H_TAIL:
        pltpu.sync_copy(edges_hbm.at[1, pl.ds(base + off, tl)],
                        idx_v.at[0, pl.ds(0, tl)])
        pltpu.sync_copy(ones_v.at[pl.ds(0, tl)],
                        deg_sp.at[idx_v.at[0, pl.ds(0, tl)]], add=True)
        off += tl
    plsc.subcore_barrier()
    pltpu.sync_copy(deg_sp.at[pl.ds(s * ROWS_PER_TILE, ROWS_PER_TILE)],
                    degp_hbm.at[c, pl.ds(s * ROWS_PER_TILE, ROWS_PER_TILE)])


KA = 3
NBUF = 4
ACHUNK = KA * LANE
P_AGG = 65
A_TAIL = (128, 32)


@functools.partial(
    pl.kernel,
    out_type=jax.ShapeDtypeStruct((NC, N_PAD, D), jnp.float32),
    mesh=_mesh,
    scratch_types=[
        pltpu.VMEM_SHARED((N_PAD, D), jnp.float32),
        pltpu.VMEM((NBUF, 2, ACHUNK), jnp.int32),
        pltpu.VMEM((NBUF, ACHUNK, D), jnp.float32),
        pltpu.SemaphoreType.DMA,
        pltpu.SemaphoreType.DMA,
    ],
    compiler_params=pltpu.CompilerParams(use_tc_tiling_on_sc=False),
)
def _sc_aggregate(edges_hbm, t_hbm, zeros_hbm, aggp_hbm,
                  agg_sp, eidx, rows, gsem, ssem):
    c = lax.axis_index("c")
    s = lax.axis_index("s")
    wid = s * NC + c
    pltpu.sync_copy(zeros_hbm.at[pl.ds(s * ROWS_PER_TILE, ROWS_PER_TILE)],
                    agg_sp.at[pl.ds(s * ROWS_PER_TILE, ROWS_PER_TILE)])
    plsc.subcore_barrier()

    base = wid * E_PER_TILE

    def stage(b, ch):
        pltpu.sync_copy(edges_hbm.at[:, pl.ds(base + ch * ACHUNK, ACHUNK)],
                        eidx.at[b])

    def fire_gather(b):
        pltpu.async_copy(t_hbm.at[eidx.at[b, 0]], rows.at[b], gsem)

    def drain_gather(b):
        pltpu.make_async_copy(t_hbm.at[eidx.at[b, 0]], rows.at[b],
                              gsem).wait()

    def fire_scatter(b):
        pltpu.async_copy(rows.at[b], agg_sp.at[eidx.at[b, 1]], ssem,
                         add=True)

    def drain_scatter(b):
        pltpu.make_async_copy(rows.at[b], agg_sp.at[eidx.at[b, 1]],
                              ssem).wait()

    for b in range(NBUF):
        stage(b, b)
        fire_gather(b)

    def body(p, carry):
        for b in range(NBUF):
            drain_gather(b)
            fire_scatter(b)

        @pl.when(p + 1 < P_AGG)
        def _():
            for b in range(NBUF):
                drain_scatter(b)
                stage(b, (p + 1) * NBUF + b)
                fire_gather(b)
        return carry

    lax.fori_loop(0, P_AGG, body, 0)
    for b in range(NBUF):
        drain_scatter(b)
    off = P_AGG * NBUF * ACHUNK
    for tl in A_TAIL:
        pltpu.sync_copy(edges_hbm.at[:, pl.ds(base + off, tl)],
                        eidx.at[0, :, pl.ds(0, tl)])
        pltpu.sync_copy(t_hbm.at[eidx.at[0, 0, pl.ds(0, tl)]],
                        rows.at[0, pl.ds(0, tl)])
        pltpu.sync_copy(rows.at[0, pl.ds(0, tl)],
                        agg_sp.at[eidx.at[0, 1, pl.ds(0, tl)]], add=True)
        off += tl
    plsc.subcore_barrier()
    pltpu.sync_copy(agg_sp.at[pl.ds(s * ROWS_PER_TILE, ROWS_PER_TILE)],
                    aggp_hbm.at[c, pl.ds(s * ROWS_PER_TILE, ROWS_PER_TILE)])


PR = N_PAD // 8
PB = BLK // 8


def _tc_scale_body(dexp_ref, x_ref, t_ref):
    dis = lax.rsqrt(dexp_ref[...] + 1.0)
    t_ref[...] = x_ref[...] * dis


def _tc_scale(dexp, x128):
    return pl.pallas_call(
        _tc_scale_body,
        grid=(GRID,),
        in_specs=[
            pl.BlockSpec((PB, LANE), lambda i: (i, 0)),
            pl.BlockSpec((PB, LANE), lambda i: (i, 0)),
        ],
        out_specs=pl.BlockSpec((PB, LANE), lambda i: (i, 0)),
        out_shape=jax.ShapeDtypeStruct((PR, LANE), jnp.float32),
    )(dexp, x128)


def _tc_finish_body(aggp_ref, t_ref, dexp_ref, wbd_ref, b_ref, g_ref,
                    out_ref, z_ref):
    dis = lax.rsqrt(dexp_ref[...] + 1.0)
    h = (aggp_ref[0] + aggp_ref[1] + t_ref[...]) * dis
    o = jnp.dot(h, wbd_ref[...], preferred_element_type=jnp.float32) \
        + b_ref[...]
    m = jnp.max(o, axis=1, keepdims=True)
    e = jnp.exp(o - m)
    sums = jnp.dot(e, g_ref[...], preferred_element_type=jnp.float32)
    out_ref[...] = o
    z_ref[...] = o - m - jnp.log(sums)


def _tc_finish(aggp128, t128, dexp, wbd, b128, g):
    return pl.pallas_call(
        _tc_finish_body,
        grid=(GRID,),
        in_specs=[
            pl.BlockSpec((NC, PB, LANE), lambda i: (0, i, 0)),
            pl.BlockSpec((PB, LANE), lambda i: (i, 0)),
            pl.BlockSpec((PB, LANE), lambda i: (i, 0)),
            pl.BlockSpec((LANE, LANE), lambda i: (0, 0)),
            pl.BlockSpec((1, LANE), lambda i: (0, 0)),
            pl.BlockSpec((LANE, LANE), lambda i: (0, 0)),
        ],
        out_specs=[
            pl.BlockSpec((PB, LANE), lambda i: (i, 0)),
            pl.BlockSpec((PB, LANE), lambda i: (i, 0)),
        ],
        out_shape=[
            jax.ShapeDtypeStruct((N // 8, LANE), jnp.float32),
            jax.ShapeDtypeStruct((N // 8, LANE), jnp.float32),
        ],
    )(aggp128, t128, dexp, wbd, b128, g)


def kernel(x, edge_index, W, b):
    degp = _sc_degree(edge_index)
    dexp = jnp.repeat(degp[0] + degp[1], D).reshape(PR, LANE)
    t128 = _tc_scale(dexp, x.reshape(N // 8, LANE))
    aggp = _sc_aggregate(edge_index, t128.reshape(N_PAD, D),
                         jnp.zeros((N_PAD, D), jnp.float32))
    wbd = jnp.kron(jnp.eye(8, dtype=jnp.float32), W)
    g = jnp.kron(jnp.eye(8, dtype=jnp.float32),
                 jnp.ones((D, D), jnp.float32))
    b128 = jnp.tile(b, 8).reshape(1, LANE)
    out128, z128 = _tc_finish(aggp.reshape(NC, PR, LANE), t128, dexp, wbd,
                              b128, g)
    return (out128.reshape(N, D), z128.reshape(N, D))


# speedup vs baseline: 1.0436x; 1.0436x over previous
import functools

import jax
import jax.numpy as jnp
from jax import lax
from jax.experimental import pallas as pl
from jax.experimental.pallas import tpu as pltpu
from jax.experimental.pallas import tpu_sc as plsc

N = 100000
E = 3200000
D = 16

NC = 2
NS = 16
NW = NC * NS

LANE = 128
N_PAD = 100352
ROWS_PER_TILE = N_PAD // NS
E_PER_TILE = E // NW
HCHUNK = 1024
H_PAIRS = 48
H_TAIL = (1024, 512, 128, 32)
BLK = 2048
GRID = N_PAD // BLK

_mesh = plsc.VectorSubcoreMesh(
    core_axis_name="c", subcore_axis_name="s", num_cores=NC, num_subcores=NS)


@functools.partial(
    pl.kernel,
    out_type=jax.ShapeDtypeStruct((NC, N_PAD), jnp.float32),
    mesh=_mesh,
    scratch_types=[
        pltpu.VMEM_SHARED((N_PAD,), jnp.float32),
        pltpu.VMEM((2, HCHUNK), jnp.int32),
        pltpu.VMEM((HCHUNK,), jnp.float32),
        pltpu.VMEM((ROWS_PER_TILE // 8,), jnp.float32),
        pltpu.SemaphoreType.DMA,
    ],
    compiler_params=pltpu.CompilerParams(use_tc_tiling_on_sc=False),
)
def _sc_degree(edges_hbm, degp_hbm, deg_sp, idx_v, ones_v, zbuf, ssem):
    c = lax.axis_index("c")
    s = lax.axis_index("s")
    wid = s * NC + c
    zc = ROWS_PER_TILE // 8

    def zfill(i, carry):
        zbuf[pl.ds(i * 16, 16)] = jnp.zeros((16,), jnp.float32)
        return carry

    lax.fori_loop(0, zc // 16, zfill, 0)

    def ofill(i, carry):
        ones_v[pl.ds(i * 16, 16)] = jnp.ones((16,), jnp.float32)
        return carry

    lax.fori_loop(0, HCHUNK // 16, ofill, 0)
    for q in range(8):
        pltpu.sync_copy(zbuf,
                        deg_sp.at[pl.ds(s * ROWS_PER_TILE + q * zc, zc)])
    plsc.subcore_barrier()

    base = wid * E_PER_TILE

    def stage(b, ch):
        pltpu.sync_copy(edges_hbm.at[1, pl.ds(base + ch * HCHUNK, HCHUNK)],
                        idx_v.at[b])

    def fire(b):
        pltpu.async_copy(ones_v, deg_sp.at[idx_v.at[b]], ssem, add=True)

    def drain(b):
        pltpu.make_async_copy(ones_v, deg_sp.at[idx_v.at[b]], ssem).wait()

    stage(0, 0)

    def body(p, carry):
        fire(0)
        stage(1, 2 * p + 1)
        drain(0)
        fire(1)

        @pl.when(p + 1 < H_PAIRS)
        def _():
            stage(0, 2 * p + 2)
        drain(1)
        return carry

    lax.fori_loop(0, H_PAIRS, body, 0)
    off = 2 * H_PAIRS * HCHUNK
    for tl in H_TAIL:
        pltpu.sync_copy(edges_hbm.at[1, pl.ds(base + off, tl)],
                        idx_v.at[0, pl.ds(0, tl)])
        pltpu.sync_copy(ones_v.at[pl.ds(0, tl)],
                        deg_sp.at[idx_v.at[0, pl.ds(0, tl)]], add=True)
        off += tl
    plsc.subcore_barrier()
    pltpu.sync_copy(deg_sp.at[pl.ds(s * ROWS_PER_TILE, ROWS_PER_TILE)],
                    degp_hbm.at[c, pl.ds(s * ROWS_PER_TILE, ROWS_PER_TILE)])


KA = 6
NBUF = 2
ACHUNK = KA * LANE
P_AGG = 65
A_TAIL = (128, 32)


@functools.partial(
    pl.kernel,
    out_type=jax.ShapeDtypeStruct((NC, N_PAD, D), jnp.float32),
    mesh=_mesh,
    scratch_types=[
        pltpu.VMEM_SHARED((N_PAD, D), jnp.float32),
        pltpu.VMEM((NBUF, 2, ACHUNK), jnp.int32),
        pltpu.VMEM((NBUF, ACHUNK, D), jnp.float32),
        pltpu.SemaphoreType.DMA,
        pltpu.SemaphoreType.DMA,
    ],
    compiler_params=pltpu.CompilerParams(use_tc_tiling_on_sc=False),
)
def _sc_aggregate(edges_hbm, t_hbm, zeros_hbm, aggp_hbm,
                  agg_sp, eidx, rows, gsem, ssem):
    c = lax.axis_index("c")
    s = lax.axis_index("s")
    wid = s * NC + c
    pltpu.sync_copy(zeros_hbm.at[pl.ds(s * ROWS_PER_TILE, ROWS_PER_TILE)],
                    agg_sp.at[pl.ds(s * ROWS_PER_TILE, ROWS_PER_TILE)])
    plsc.subcore_barrier()

    base = wid * E_PER_TILE

    def stage(b, ch):
        pltpu.sync_copy(edges_hbm.at[:, pl.ds(base + ch * ACHUNK, ACHUNK)],
                        eidx.at[b])

    def fire_gather(b):
        pltpu.async_copy(t_hbm.at[eidx.at[b, 0]], rows.at[b], gsem)

    def drain_gather(b):
        pltpu.make_async_copy(t_hbm.at[eidx.at[b, 0]], rows.at[b],
                              gsem).wait()

    def fire_scatter(b):
        pltpu.async_copy(rows.at[b], agg_sp.at[eidx.at[b, 1]], ssem,
                         add=True)

    def drain_scatter(b):
        pltpu.make_async_copy(rows.at[b], agg_sp.at[eidx.at[b, 1]],
                              ssem).wait()

    for b in range(NBUF):
        stage(b, b)
        fire_gather(b)

    def body(p, carry):
        for b in range(NBUF):
            drain_gather(b)
            fire_scatter(b)

        @pl.when(p + 1 < P_AGG)
        def _():
            for b in range(NBUF):
                drain_scatter(b)
                stage(b, (p + 1) * NBUF + b)
                fire_gather(b)
        return carry

    lax.fori_loop(0, P_AGG, body, 0)
    for b in range(NBUF):
        drain_scatter(b)
    off = P_AGG * NBUF * ACHUNK
    for tl in A_TAIL:
        pltpu.sync_copy(edges_hbm.at[:, pl.ds(base + off, tl)],
                        eidx.at[0, :, pl.ds(0, tl)])
        pltpu.sync_copy(t_hbm.at[eidx.at[0, 0, pl.ds(0, tl)]],
                        rows.at[0, pl.ds(0, tl)])
        pltpu.sync_copy(rows.at[0, pl.ds(0, tl)],
                        agg_sp.at[eidx.at[0, 1, pl.ds(0, tl)]], add=True)
        off += tl
    plsc.subcore_barrier()
    pltpu.sync_copy(agg_sp.at[pl.ds(s * ROWS_PER_TILE, ROWS_PER_TILE)],
                    aggp_hbm.at[c, pl.ds(s * ROWS_PER_TILE, ROWS_PER_TILE)])


PR = N_PAD // 8
PB = BLK // 8


def _tc_scale_body(dexp_ref, x_ref, t_ref):
    dis = lax.rsqrt(dexp_ref[...] + 1.0)
    t_ref[...] = x_ref[...] * dis


def _tc_scale(dexp, x128):
    return pl.pallas_call(
        _tc_scale_body,
        grid=(GRID,),
        in_specs=[
            pl.BlockSpec((PB, LANE), lambda i: (i, 0)),
            pl.BlockSpec((PB, LANE), lambda i: (i, 0)),
        ],
        out_specs=pl.BlockSpec((PB, LANE), lambda i: (i, 0)),
        out_shape=jax.ShapeDtypeStruct((PR, LANE), jnp.float32),
    )(dexp, x128)


def _tc_finish_body(aggp_ref, t_ref, dexp_ref, wbd_ref, b_ref, g_ref,
                    out_ref, z_ref):
    dis = lax.rsqrt(dexp_ref[...] + 1.0)
    h = (aggp_ref[0] + aggp_ref[1] + t_ref[...]) * dis
    o = jnp.dot(h, wbd_ref[...], preferred_element_type=jnp.float32) \
        + b_ref[...]
    m = jnp.max(o, axis=1, keepdims=True)
    e = jnp.exp(o - m)
    sums = jnp.dot(e, g_ref[...], preferred_element_type=jnp.float32)
    out_ref[...] = o
    z_ref[...] = o - m - jnp.log(sums)


def _tc_finish(aggp128, t128, dexp, wbd, b128, g):
    return pl.pallas_call(
        _tc_finish_body,
        grid=(GRID,),
        in_specs=[
            pl.BlockSpec((NC, PB, LANE), lambda i: (0, i, 0)),
            pl.BlockSpec((PB, LANE), lambda i: (i, 0)),
            pl.BlockSpec((PB, LANE), lambda i: (i, 0)),
            pl.BlockSpec((LANE, LANE), lambda i: (0, 0)),
            pl.BlockSpec((1, LANE), lambda i: (0, 0)),
            pl.BlockSpec((LANE, LANE), lambda i: (0, 0)),
        ],
        out_specs=[
            pl.BlockSpec((PB, LANE), lambda i: (i, 0)),
            pl.BlockSpec((PB, LANE), lambda i: (i, 0)),
        ],
        out_shape=[
            jax.ShapeDtypeStruct((N // 8, LANE), jnp.float32),
            jax.ShapeDtypeStruct((N // 8, LANE), jnp.float32),
        ],
    )(aggp128, t128, dexp, wbd, b128, g)


def kernel(x, edge_index, W, b):
    degp = _sc_degree(edge_index)
    dexp = jnp.repeat(degp[0] + degp[1], D).reshape(PR, LANE)
    t128 = _tc_scale(dexp, x.reshape(N // 8, LANE))
    aggp = _sc_aggregate(edge_index, t128.reshape(N_PAD, D),
                         jnp.zeros((N_PAD, D), jnp.float32))
    wbd = jnp.kron(jnp.eye(8, dtype=jnp.float32), W)
    g = jnp.kron(jnp.eye(8, dtype=jnp.float32),
                 jnp.ones((D, D), jnp.float32))
    b128 = jnp.tile(b, 8).reshape(1, LANE)
    out128, z128 = _tc_finish(aggp.reshape(NC, PR, LANE), t128, dexp, wbd,
                              b128, g)
    return (out128.reshape(N, D), z128.reshape(N, D))


# speedup vs baseline: 1.0914x; 1.0458x over previous
import functools

import jax
import jax.numpy as jnp
from jax import lax
from jax.experimental import pallas as pl
from jax.experimental.pallas import tpu as pltpu
from jax.experimental.pallas import tpu_sc as plsc

N = 100000
E = 3200000
D = 16

NC = 2
NS = 16
NW = NC * NS

LANE = 128
N_PAD = 100352
ROWS_PER_TILE = N_PAD // NS
E_PER_TILE = E // NW
HCHUNK = 2048
H_PAIRS = 24
H_TAIL = (1024, 512, 128, 32)
BLK = 2048
GRID = N_PAD // BLK

_mesh = plsc.VectorSubcoreMesh(
    core_axis_name="c", subcore_axis_name="s", num_cores=NC, num_subcores=NS)


@functools.partial(
    pl.kernel,
    out_type=jax.ShapeDtypeStruct((NC, N_PAD), jnp.float32),
    mesh=_mesh,
    scratch_types=[
        pltpu.VMEM_SHARED((N_PAD,), jnp.float32),
        pltpu.VMEM((2, HCHUNK), jnp.int32),
        pltpu.VMEM((HCHUNK,), jnp.float32),
        pltpu.VMEM((ROWS_PER_TILE // 8,), jnp.float32),
        pltpu.SemaphoreType.DMA,
    ],
    compiler_params=pltpu.CompilerParams(use_tc_tiling_on_sc=False),
)
def _sc_degree(edges_hbm, degp_hbm, deg_sp, idx_v, ones_v, zbuf, ssem):
    c = lax.axis_index("c")
    s = lax.axis_index("s")
    wid = s * NC + c
    zc = ROWS_PER_TILE // 8

    def zfill(i, carry):
        zbuf[pl.ds(i * 16, 16)] = jnp.zeros((16,), jnp.float32)
        return carry

    lax.fori_loop(0, zc // 16, zfill, 0)

    def ofill(i, carry):
        ones_v[pl.ds(i * 16, 16)] = jnp.ones((16,), jnp.float32)
        return carry

    lax.fori_loop(0, HCHUNK // 16, ofill, 0)
    for q in range(8):
        pltpu.sync_copy(zbuf,
                        deg_sp.at[pl.ds(s * ROWS_PER_TILE + q * zc, zc)])
    plsc.subcore_barrier()

    base = wid * E_PER_TILE

    def stage(b, ch):
        pltpu.sync_copy(edges_hbm.at[1, pl.ds(base + ch * HCHUNK, HCHUNK)],
                        idx_v.at[b])

    def fire(b):
        pltpu.async_copy(ones_v, deg_sp.at[idx_v.at[b]], ssem, add=True)

    def drain(b):
        pltpu.make_async_copy(ones_v, deg_sp.at[idx_v.at[b]], ssem).wait()

    stage(0, 0)

    def body(p, carry):
        fire(0)
        stage(1, 2 * p + 1)
        drain(0)
        fire(1)

        @pl.when(p + 1 < H_PAIRS)
        def _():
            stage(0, 2 * p + 2)
        drain(1)
        return carry

    lax.fori_loop(0, H_PAIRS, body, 0)
    off = 2 * H_PAIRS * HCHUNK
    for tl in H_TAIL:
        pltpu.sync_copy(edges_hbm.at[1, pl.ds(base + off, tl)],
                        idx_v.at[0, pl.ds(0, tl)])
        pltpu.sync_copy(ones_v.at[pl.ds(0, tl)],
                        deg_sp.at[idx_v.at[0, pl.ds(0, tl)]], add=True)
        off += tl
    plsc.subcore_barrier()
    pltpu.sync_copy(deg_sp.at[pl.ds(s * ROWS_PER_TILE, ROWS_PER_TILE)],
                    degp_hbm.at[c, pl.ds(s * ROWS_PER_TILE, ROWS_PER_TILE)])


KA = 6
NBUF = 2
ACHUNK = KA * LANE
P_AGG = 65
A_TAIL = (128, 32)


@functools.partial(
    pl.kernel,
    out_type=jax.ShapeDtypeStruct((NC, N_PAD, D), jnp.float32),
    mesh=_mesh,
    scratch_types=[
        pltpu.VMEM_SHARED((N_PAD, D), jnp.float32),
        pltpu.VMEM((NBUF, 2, ACHUNK), jnp.int32),
        pltpu.VMEM((NBUF, ACHUNK, D), jnp.float32),
        pltpu.SemaphoreType.DMA,
        pltpu.SemaphoreType.DMA,
        pltpu.SemaphoreType.DMA,
    ],
    compiler_params=pltpu.CompilerParams(use_tc_tiling_on_sc=False),
)
def _sc_aggregate(edges_hbm, t_hbm, zeros_hbm, aggp_hbm,
                  agg_sp, eidx, rows, gsem, ssem, stsem):
    c = lax.axis_index("c")
    s = lax.axis_index("s")
    wid = s * NC + c
    pltpu.sync_copy(zeros_hbm.at[pl.ds(s * ROWS_PER_TILE, ROWS_PER_TILE)],
                    agg_sp.at[pl.ds(s * ROWS_PER_TILE, ROWS_PER_TILE)])
    plsc.subcore_barrier()

    base = wid * E_PER_TILE

    def stage(b, ch):
        pltpu.sync_copy(edges_hbm.at[:, pl.ds(base + ch * ACHUNK, ACHUNK)],
                        eidx.at[b])

    def stage_start(b, ch):
        pltpu.async_copy(edges_hbm.at[:, pl.ds(base + ch * ACHUNK, ACHUNK)],
                         eidx.at[b], stsem)

    def stage_wait(b, ch):
        pltpu.make_async_copy(
            edges_hbm.at[:, pl.ds(base + ch * ACHUNK, ACHUNK)],
            eidx.at[b], stsem).wait()

    def fire_gather(b):
        pltpu.async_copy(t_hbm.at[eidx.at[b, 0]], rows.at[b], gsem)

    def drain_gather(b):
        pltpu.make_async_copy(t_hbm.at[eidx.at[b, 0]], rows.at[b],
                              gsem).wait()

    def fire_scatter(b):
        pltpu.async_copy(rows.at[b], agg_sp.at[eidx.at[b, 1]], ssem,
                         add=True)

    def drain_scatter(b):
        pltpu.make_async_copy(rows.at[b], agg_sp.at[eidx.at[b, 1]],
                              ssem).wait()

    for b in range(NBUF):
        stage(b, b)
        fire_gather(b)

    def body(p, carry):
        for b in range(NBUF):
            drain_gather(b)
            fire_scatter(b)

        @pl.when(p + 1 < P_AGG)
        def _():
            for b in range(NBUF):
                drain_scatter(b)
                stage_start(b, (p + 1) * NBUF + b)
            for b in range(NBUF):
                stage_wait(b, (p + 1) * NBUF + b)
                fire_gather(b)
        return carry

    lax.fori_loop(0, P_AGG, body, 0)
    for b in range(NBUF):
        drain_scatter(b)
    off = P_AGG * NBUF * ACHUNK
    for tl in A_TAIL:
        pltpu.sync_copy(edges_hbm.at[:, pl.ds(base + off, tl)],
                        eidx.at[0, :, pl.ds(0, tl)])
        pltpu.sync_copy(t_hbm.at[eidx.at[0, 0, pl.ds(0, tl)]],
                        rows.at[0, pl.ds(0, tl)])
        pltpu.sync_copy(rows.at[0, pl.ds(0, tl)],
                        agg_sp.at[eidx.at[0, 1, pl.ds(0, tl)]], add=True)
        off += tl
    plsc.subcore_barrier()
    pltpu.sync_copy(agg_sp.at[pl.ds(s * ROWS_PER_TILE, ROWS_PER_TILE)],
                    aggp_hbm.at[c, pl.ds(s * ROWS_PER_TILE, ROWS_PER_TILE)])


PR = N_PAD // 8
PB = BLK // 8


def _tc_scale_body(dexp_ref, x_ref, t_ref):
    dis = lax.rsqrt(dexp_ref[...] + 1.0)
    t_ref[...] = x_ref[...] * dis


def _tc_scale(dexp, x128):
    return pl.pallas_call(
        _tc_scale_body,
        grid=(GRID,),
        in_specs=[
            pl.BlockSpec((PB, LANE), lambda i: (i, 0)),
            pl.BlockSpec((PB, LANE), lambda i: (i, 0)),
        ],
        out_specs=pl.BlockSpec((PB, LANE), lambda i: (i, 0)),
        out_shape=jax.ShapeDtypeStruct((PR, LANE), jnp.float32),
    )(dexp, x128)


def _tc_finish_body(aggp_ref, t_ref, dexp_ref, wbd_ref, b_ref, g_ref,
                    out_ref, z_ref):
    dis = lax.rsqrt(dexp_ref[...] + 1.0)
    h = (aggp_ref[0] + aggp_ref[1] + t_ref[...]) * dis
    o = jnp.dot(h, wbd_ref[...], preferred_element_type=jnp.float32) \
        + b_ref[...]
    m = jnp.max(o, axis=1, keepdims=True)
    e = jnp.exp(o - m)
    sums = jnp.dot(e, g_ref[...], preferred_element_type=jnp.float32)
    out_ref[...] = o
    z_ref[...] = o - m - jnp.log(sums)


def _tc_finish(aggp128, t128, dexp, wbd, b128, g):
    return pl.pallas_call(
        _tc_finish_body,
        grid=(GRID,),
        in_specs=[
            pl.BlockSpec((NC, PB, LANE), lambda i: (0, i, 0)),
            pl.BlockSpec((PB, LANE), lambda i: (i, 0)),
            pl.BlockSpec((PB, LANE), lambda i: (i, 0)),
            pl.BlockSpec((LANE, LANE), lambda i: (0, 0)),
            pl.BlockSpec((1, LANE), lambda i: (0, 0)),
            pl.BlockSpec((LANE, LANE), lambda i: (0, 0)),
        ],
        out_specs=[
            pl.BlockSpec((PB, LANE), lambda i: (i, 0)),
            pl.BlockSpec((PB, LANE), lambda i: (i, 0)),
        ],
        out_shape=[
            jax.ShapeDtypeStruct((N // 8, LANE), jnp.float32),
            jax.ShapeDtypeStruct((N // 8, LANE), jnp.float32),
        ],
    )(aggp128, t128, dexp, wbd, b128, g)


def kernel(x, edge_index, W, b):
    degp = _sc_degree(edge_index)
    dexp = jnp.repeat(degp[0] + degp[1], D).reshape(PR, LANE)
    t128 = _tc_scale(dexp, x.reshape(N // 8, LANE))
    aggp = _sc_aggregate(edge_index, t128.reshape(N_PAD, D),
                         jnp.zeros((N_PAD, D), jnp.float32))
    wbd = jnp.kron(jnp.eye(8, dtype=jnp.float32), W)
    g = jnp.kron(jnp.eye(8, dtype=jnp.float32),
                 jnp.ones((D, D), jnp.float32))
    b128 = jnp.tile(b, 8).reshape(1, LANE)
    out128, z128 = _tc_finish(aggp.reshape(NC, PR, LANE), t128, dexp, wbd,
                              b128, g)
    return (out128.reshape(N, D), z128.reshape(N, D))
